# baseline (speedup 1.0000x reference)
import jax
import jax.numpy as jnp
from jax import lax
from jax.experimental import pallas as pl
from jax.experimental.pallas import tpu as pltpu
from jax.experimental.pallas import tpu_sc as plsc

NC = 2
NS = 16

CH = 128
N_NODES = 10000
N_ACC = 10240
D_IN = 128
D_HALF = D_IN // NC
D_EDGE = 16
E_HALF = D_EDGE // NC
E_EDGES = 320000
N_CHUNKS = E_EDGES // CH
T_CH = 160
BLK = 16
NB = 2


def _sc_body(node_hbm, src_hbm, dst_hbm, ef_hbm, a_out, b_out,
             a_acc, b_acc, src_vb, dst_vb, rows2, ef2,
             gsem, esem, sasem, sbsem):
    cid = lax.axis_index("c")
    sid = lax.axis_index("s")

    zv = jnp.zeros((16,), jnp.float32)
    cpr = D_HALF // 16

    def _zrow(i, c):
        rows2[0, i // cpr, pl.ds((i % cpr) * 16, 16)] = zv
        return c

    lax.fori_loop(0, (CH * D_HALF) // 16, _zrow, 0)

    for z in range(N_ACC // NS // CH):
        base = sid * (N_ACC // NS) + z * CH
        pltpu.sync_copy(rows2.at[0], a_acc.at[pl.ds(base, CH)])
        pltpu.sync_copy(rows2.at[0, :, pl.ds(0, E_HALF)],
                        b_acc.at[pl.ds(base, CH)])

    plsc.subcore_barrier()

    gbase = sid * T_CH

    def _blk(bi, c):
        base_ch = bi * BLK
        pltpu.sync_copy(src_hbm.at[sid, pl.ds(base_ch, BLK)], src_vb)
        pltpu.sync_copy(dst_hbm.at[sid, pl.ds(base_ch, BLK)], dst_vb)

        def fire(j):
            b = j % NB
            g = gbase + base_ch + j
            pltpu.async_copy(
                node_hbm.at[cid].at[src_vb.at[j]], rows2.at[b], gsem.at[b])

            @pl.when(g < N_CHUNKS)
            def _():
                pltpu.async_copy(
                    ef_hbm.at[cid, g], ef2.at[b], esem.at[b])

        fire(0)
        for j in range(BLK):
            b = j % NB
            g = gbase + base_ch + j
            if j + 1 < BLK:
                fire(j + 1)
            pltpu.make_async_copy(
                node_hbm.at[cid].at[src_vb.at[j]], rows2.at[b],
                gsem.at[b]).wait()

            @pl.when(g < N_CHUNKS)
            def _():
                pltpu.make_async_copy(
                    ef_hbm.at[cid, g], ef2.at[b], esem.at[b]).wait()

            pltpu.async_copy(
                rows2.at[b], a_acc.at[dst_vb.at[j]], sasem.at[b], add=True)

            @pl.when(g < N_CHUNKS)
            def _():
                pltpu.async_copy(
                    ef2.at[b], b_acc.at[dst_vb.at[j]], sbsem.at[b], add=True)

            pltpu.make_async_copy(
                rows2.at[b], a_acc.at[dst_vb.at[j]], sasem.at[b]).wait()

            @pl.when(g < N_CHUNKS)
            def _():
                pltpu.make_async_copy(
                    ef2.at[b], b_acc.at[dst_vb.at[j]], sbsem.at[b]).wait()
        return c

    lax.fori_loop(0, T_CH // BLK, _blk, 0)

    plsc.subcore_barrier()

    out_rows = N_ACC // NS
    obase = sid * out_rows
    pltpu.sync_copy(a_acc.at[pl.ds(obase, out_rows)],
                    a_out.at[cid, pl.ds(obase, out_rows)])
    pltpu.sync_copy(b_acc.at[pl.ds(obase, out_rows)],
                    b_out.at[cid, pl.ds(obase, out_rows)])


_sc_call = pl.kernel(
    _sc_body,
    out_type=(
        jax.ShapeDtypeStruct((NC, N_ACC, D_HALF), jnp.float32),
        jax.ShapeDtypeStruct((NC, N_ACC, E_HALF), jnp.float32),
    ),
    mesh=plsc.VectorSubcoreMesh(
        core_axis_name="c", subcore_axis_name="s",
        num_cores=NC, num_subcores=NS),
    compiler_params=pltpu.CompilerParams(use_tc_tiling_on_sc=False),
    scratch_types=[
        pltpu.VMEM_SHARED((N_ACC, D_HALF), jnp.float32),
        pltpu.VMEM_SHARED((N_ACC, E_HALF), jnp.float32),
        pltpu.VMEM((BLK, CH), jnp.int32),
        pltpu.VMEM((BLK, CH), jnp.int32),
        pltpu.VMEM((NB, CH, D_HALF), jnp.float32),
        pltpu.VMEM((NB, CH, E_HALF), jnp.float32),
        pltpu.SemaphoreType.DMA((NB,)),
        pltpu.SemaphoreType.DMA((NB,)),
        pltpu.SemaphoreType.DMA((NB,)),
        pltpu.SemaphoreType.DMA((NB,)),
    ],
)


def _mm_body(a_ref, b_ref, w1_ref, w2_ref, o_ref):
    a = jnp.concatenate([a_ref[0], a_ref[1]], axis=-1)
    b = jnp.concatenate([b_ref[0], b_ref[1]], axis=-1)
    o_ref[...] = (
        jnp.dot(a, w1_ref[...], preferred_element_type=jnp.float32)
        + jnp.dot(b, w2_ref[...], preferred_element_type=jnp.float32))


def _combine(A, B, W1, W2):
    blk = 1000
    grid = (N_NODES // blk,)
    return pl.pallas_call(
        _mm_body,
        grid=grid,
        in_specs=[
            pl.BlockSpec((NC, blk, D_HALF), lambda i: (0, i, 0)),
            pl.BlockSpec((NC, blk, E_HALF), lambda i: (0, i, 0)),
            pl.BlockSpec((D_IN, D_IN), lambda i: (0, 0)),
            pl.BlockSpec((D_EDGE, D_IN), lambda i: (0, 0)),
        ],
        out_specs=pl.BlockSpec((blk, D_IN), lambda i: (i, 0)),
        out_shape=jax.ShapeDtypeStruct((N_NODES, D_IN), jnp.float32),
    )(A, B, W1, W2)


def kernel(node_feature, edge_index, edge_feature, W):
    N, D = node_feature.shape
    E = edge_index.shape[1]
    assert (N, D, E) == (N_NODES, D_IN, E_EDGES)

    keep = jax.random.uniform(jax.random.key(42), (E,)) < 0.5
    src = edge_index[0]
    dst = jnp.where(keep, edge_index[1], N).astype(jnp.int32)

    node_half = node_feature.reshape(N, NC, D_HALF).transpose(1, 0, 2)
    ef_half = edge_feature.reshape(N_CHUNKS, CH, NC, E_HALF).transpose(
        2, 0, 1, 3)

    pad = NS * T_CH * CH - E_EDGES
    src_p = jnp.concatenate(
        [src, jnp.zeros((pad,), jnp.int32)]).reshape(NS, T_CH, CH)
    dst_p = jnp.concatenate(
        [dst, jnp.full((pad,), N_NODES, jnp.int32)]).reshape(NS, T_CH, CH)

    A, B = _sc_call(node_half, src_p, dst_p, ef_half)
    return _combine(A, B, W[:D], W[D:])
